# baseline (speedup 1.0000x reference)
import functools

import jax
import jax.numpy as jnp
import numpy as np
from jax import lax
from jax.experimental import pallas as pl
from jax.experimental.pallas import tpu as pltpu
from jax.experimental.pallas import tpu_sc as plsc

VOCAB = 1000000
D = 64
BATCH = 4096
SEQ = 200

NC = 2
NS = 16
NW = NC * NS
LANES = 16

BT = BATCH // NW
DPAIRS = D // 2
DT = D // 8


def _compute_encoding(max_len, d):
    enc = np.zeros((max_len, d), dtype=np.float32)
    pos = np.arange(0, max_len, dtype=np.float32)
    for i in range(d // 2):
        enc[:, 2 * i] = np.sin(pos / 10000 ** (2 * i / d))
        enc[:, 2 * i + 1] = np.cos(pos / 10000 ** (2 * i / d))
    return enc


_ENC = _compute_encoding(SEQ, D)


def _body(xt_hbm, tab_hbm, enc_hbm, out_hbm,
          idx_v, emb_v, out_v, enc_v, idx_sem, gat_sem, out_sem):
    w = lax.axis_index("s") * NC + lax.axis_index("c")

    pltpu.sync_copy(enc_hbm, enc_v)

    def idx_start(s, b):
        pltpu.make_async_copy(
            xt_hbm.at[s, pl.ds(w * BT, BT)], idx_v.at[b], idx_sem.at[b]).start()

    def idx_wait(b):
        pltpu.make_async_copy(
            xt_hbm.at[0, pl.ds(0, BT)], idx_v.at[b], idx_sem.at[b]).wait()

    def gather_start(b):
        pltpu.make_async_copy(
            tab_hbm.at[idx_v.at[b]], emb_v.at[b], gat_sem.at[b]).start()

    def gather_wait(b):
        pltpu.make_async_copy(
            tab_hbm.at[pl.ds(0, BT)], emb_v.at[b], gat_sem.at[b]).wait()

    def out_start(s, b):
        pltpu.make_async_copy(
            out_v.at[b], out_hbm.at[s, :, w], out_sem.at[b]).start()

    def out_wait(b):
        pltpu.make_async_copy(
            out_v.at[b], out_hbm.at[0, :, 0], out_sem.at[b]).wait()

    iota = lax.iota(jnp.int32, LANES)

    def compute(s, b):
        evecs = [enc_v[s, pl.ds(16 * j, 16)] for j in range(D // LANES)]
        for dt in range(DT):
            for k in range(DPAIRS // DT):
                col = dt * (DPAIRS // DT) + k
                d0 = 2 * col
                e0 = evecs[d0 // LANES][d0 % LANES]
                e1 = evecs[d0 // LANES][d0 % LANES + 1]
                cols = jnp.full((LANES,), col, dtype=jnp.int32)
                for lb in range(BT // LANES):
                    rows = lb * LANES + iota
                    pair = plsc.load_gather(emb_v.at[b], [rows, cols])
                    bf = plsc.bitcast(pair, jnp.bfloat16)
                    lo, hi = plsc.unpack(
                        bf, format=plsc.PackFormat.INTERLEAVED,
                        preferred_element_type=jnp.float32)
                    sl = pl.ds(lb * LANES, LANES)
                    out_v[b, dt, 2 * k, sl] = lo + e0
                    out_v[b, dt, 2 * k + 1, sl] = hi + e1

    idx_start(0, 0)
    idx_wait(0)
    gather_start(0)
    idx_start(1, 1)

    def pair_body(s2, _):
        for b in (0, 1):
            s = 2 * s2 + b
            nb = 1 - b

            @pl.when(s >= 2)
            def _():
                out_wait(b)

            gather_wait(b)

            @pl.when(s + 1 < SEQ)
            def _():
                idx_wait(nb)
                gather_start(nb)

            @pl.when(s + 2 < SEQ)
            def _():
                idx_start(s + 2, b)

            compute(s, b)
            out_start(s, b)
        return ()

    lax.fori_loop(0, SEQ // 2, pair_body, (), unroll=False)

    out_wait(0)
    out_wait(1)


@jax.jit
def kernel(x, table):
    xt = x.T
    tab_bf = table.astype(jnp.bfloat16)
    lo = lax.bitcast_convert_type(tab_bf[:, 0::2], jnp.uint16).astype(jnp.uint32)
    hi = lax.bitcast_convert_type(tab_bf[:, 1::2], jnp.uint16).astype(jnp.uint32)
    tab_pairs = lax.bitcast_convert_type(
        (hi << 16) | lo, jnp.float32)
    enc = jnp.asarray(_ENC)
    mesh = plsc.VectorSubcoreMesh(core_axis_name="c", subcore_axis_name="s")
    out5d = pl.kernel(
        _body,
        out_type=jax.ShapeDtypeStruct((SEQ, DT, NW, 8, BT), jnp.float32),
        mesh=mesh,
        compiler_params=pltpu.CompilerParams(
            use_tc_tiling_on_sc=False, needs_layout_passes=False),
        scratch_types=[
            pltpu.VMEM((2, BT), jnp.int32),
            pltpu.VMEM((2, BT, DPAIRS), jnp.float32),
            pltpu.VMEM((2, DT, 8, BT), jnp.float32),
            pltpu.VMEM((SEQ, D), jnp.float32),
            pltpu.SemaphoreType.DMA((2,)),
            pltpu.SemaphoreType.DMA((2,)),
            pltpu.SemaphoreType.DMA((2,)),
        ],
    )(xt, tab_pairs, enc)
    return out5d.transpose(2, 4, 0, 1, 3).reshape(BATCH, SEQ, D)


# speedup vs baseline: 1.3681x; 1.3681x over previous
import functools

import jax
import jax.numpy as jnp
import numpy as np
from jax import lax
from jax.experimental import pallas as pl
from jax.experimental.pallas import tpu as pltpu
from jax.experimental.pallas import tpu_sc as plsc

VOCAB = 1000000
D = 64
BATCH = 4096
SEQ = 200

NC = 2
NS = 16
NW = NC * NS
LANES = 16

BT = BATCH // NW
DT = D // 8
LB = BT // LANES


def _compute_encoding(max_len, d):
    enc = np.zeros((max_len, d), dtype=np.float32)
    pos = np.arange(0, max_len, dtype=np.float32)
    for i in range(d // 2):
        enc[:, 2 * i] = np.sin(pos / 10000 ** (2 * i / d))
        enc[:, 2 * i + 1] = np.cos(pos / 10000 ** (2 * i / d))
    return enc


_ENC = _compute_encoding(SEQ, D)


def _body(xt_hbm, tab_hbm, enc_hbm, out_hbm,
          idx_v, idx2_v, emb_v, out_v, enc_v, idx_sem, gat_sem, out_sem):
    w = lax.axis_index("s") * NC + lax.axis_index("c")

    pltpu.sync_copy(enc_hbm, enc_v)

    def idx_start(s, b):
        pltpu.make_async_copy(
            xt_hbm.at[s, pl.ds(w * BT, BT)], idx_v.at[b], idx_sem.at[b]).start()

    def idx_wait(b):
        pltpu.make_async_copy(
            xt_hbm.at[0, pl.ds(0, BT)], idx_v.at[b], idx_sem.at[b]).wait()

    def halve_idx(b):
        for lb in range(LB):
            sl = pl.ds(lb * LANES, LANES)
            idx2_v[b, sl] = lax.shift_right_logical(idx_v[b, sl], 1)

    def gather_start(b):
        pltpu.make_async_copy(
            tab_hbm.at[idx2_v.at[b]], emb_v.at[b], gat_sem.at[b]).start()

    def gather_wait(b):
        pltpu.make_async_copy(
            tab_hbm.at[pl.ds(0, BT)], emb_v.at[b], gat_sem.at[b]).wait()

    def out_start(s, b):
        pltpu.make_async_copy(
            out_v.at[b], out_hbm.at[s, :, w], out_sem.at[b]).start()

    def out_wait(b):
        pltpu.make_async_copy(
            out_v.at[b], out_hbm.at[0, :, 0], out_sem.at[b]).wait()

    iota = lax.iota(jnp.int32, LANES)
    one = jnp.int32(1)
    six = jnp.int32(6)

    def compute(s, b):
        evecs = [enc_v[s, pl.ds(16 * j, 16)] for j in range(D // LANES)]
        rows = [lb * LANES + iota for lb in range(LB)]
        pars = [
            lax.shift_left(
                lax.bitwise_and(idx_v[b, pl.ds(lb * LANES, LANES)], one), six)
            for lb in range(LB)
        ]
        for dt in range(DT):
            for d8 in range(8):
                d = dt * 8 + d8
                e = evecs[d // LANES][d % LANES]
                for lb in range(LB):
                    g = plsc.load_gather(emb_v.at[b], [rows[lb], pars[lb] + d])
                    out_v[b, dt, d8, pl.ds(lb * LANES, LANES)] = g + e

    idx_start(0, 0)
    idx_wait(0)
    halve_idx(0)
    gather_start(0)
    idx_start(1, 1)

    def pair_body(s2, _):
        for b in (0, 1):
            s = 2 * s2 + b
            nb = 1 - b

            @pl.when(s >= 2)
            def _():
                out_wait(b)

            gather_wait(b)

            @pl.when(s + 1 < SEQ)
            def _():
                idx_wait(nb)
                halve_idx(nb)
                gather_start(nb)

            @pl.when(s + 2 < SEQ)
            def _():
                idx_start(s + 2, b)

            compute(s, b)
            out_start(s, b)
        return ()

    lax.fori_loop(0, SEQ // 2, pair_body, (), unroll=False)

    out_wait(0)
    out_wait(1)


@jax.jit
def kernel(x, table):
    xt = x.T
    tab2 = table.reshape(VOCAB // 2, 2 * D)
    enc = jnp.asarray(_ENC)
    mesh = plsc.VectorSubcoreMesh(core_axis_name="c", subcore_axis_name="s")
    out5d = pl.kernel(
        _body,
        out_type=jax.ShapeDtypeStruct((SEQ, DT, NW, 8, BT), jnp.float32),
        mesh=mesh,
        compiler_params=pltpu.CompilerParams(
            use_tc_tiling_on_sc=True, needs_layout_passes=False),
        scratch_types=[
            pltpu.VMEM((2, BT), jnp.int32),
            pltpu.VMEM((2, BT), jnp.int32),
            pltpu.VMEM((2, BT, 2 * D), jnp.float32),
            pltpu.VMEM((2, DT, 8, BT), jnp.float32),
            pltpu.VMEM((SEQ, D), jnp.float32),
            pltpu.SemaphoreType.DMA((2,)),
            pltpu.SemaphoreType.DMA((2,)),
            pltpu.SemaphoreType.DMA((2,)),
        ],
    )(xt, tab2, enc)
    return out5d.transpose(2, 4, 0, 1, 3).reshape(BATCH, SEQ, D)


# speedup vs baseline: 2.2918x; 1.6751x over previous
import functools

import jax
import jax.numpy as jnp
import numpy as np
from jax import lax
from jax.experimental import pallas as pl
from jax.experimental.pallas import tpu as pltpu
from jax.experimental.pallas import tpu_sc as plsc

VOCAB = 1000000
D = 64
BATCH = 4096
SEQ = 200
TOKENS = BATCH * SEQ

NC = 2
NS = 16
NW = NC * NS
LANES = 16

TOK_PER_W = TOKENS // NW
SEQ_PER_CHUNK = 2
CHUNK = SEQ_PER_CHUNK * SEQ
N_CHUNKS = TOK_PER_W // CHUNK
IDX_MINOR = 100
IDX_ROWS = CHUNK // IDX_MINOR
X_ROWS = TOKENS // IDX_MINOR


def _compute_encoding(max_len, d):
    enc = np.zeros((max_len, d), dtype=np.float32)
    pos = np.arange(0, max_len, dtype=np.float32)
    for i in range(d // 2):
        enc[:, 2 * i] = np.sin(pos / 10000 ** (2 * i / d))
        enc[:, 2 * i + 1] = np.cos(pos / 10000 ** (2 * i / d))
    return enc


_ENC = _compute_encoding(SEQ, D)


def _body(x_hbm, table_hbm, enc_hbm, out_hbm,
          idx_v, rows_v, enc_v, idx_sem, gat_sem, out_sem):
    wid = lax.axis_index("s") * NC + lax.axis_index("c")
    xrow0 = wid * (TOK_PER_W // IDX_MINOR)
    tok0 = wid * TOK_PER_W

    pltpu.sync_copy(enc_hbm, enc_v)

    def idx_start(c, b):
        pltpu.make_async_copy(
            x_hbm.at[pl.ds(xrow0 + c * IDX_ROWS, IDX_ROWS)],
            idx_v.at[b], idx_sem.at[b]).start()

    def idx_wait(b):
        pltpu.make_async_copy(
            x_hbm.at[pl.ds(0, IDX_ROWS)], idx_v.at[b], idx_sem.at[b]).wait()

    def gather_start(b):
        for g in range(IDX_ROWS):
            pltpu.make_async_copy(
                table_hbm.at[idx_v.at[b].at[g]],
                rows_v.at[b].at[pl.ds(g * IDX_MINOR, IDX_MINOR)],
                gat_sem.at[b]).start()

    def gather_wait(b):
        pltpu.make_async_copy(
            out_hbm.at[pl.ds(0, CHUNK)], rows_v.at[b], gat_sem.at[b]).wait()

    def out_start(c, b):
        pltpu.make_async_copy(
            rows_v.at[b], out_hbm.at[pl.ds(tok0 + c * CHUNK, CHUNK)],
            out_sem.at[b]).start()

    def out_wait(b):
        pltpu.make_async_copy(
            rows_v.at[b], out_hbm.at[pl.ds(0, CHUNK)], out_sem.at[b]).wait()

    idx_start(0, 0)
    idx_wait(0)
    gather_start(0)
    idx_start(1, 1)

    def pair_body(c2, _):
        for b in (0, 1):
            c = 2 * c2 + b
            nb = 1 - b

            @pl.when(c >= 1)
            def _():
                out_wait(nb)

            @pl.when(c + 1 < N_CHUNKS)
            def _():
                idx_wait(nb)
                gather_start(nb)

            gather_wait(b)

            @pl.when(c + 2 < N_CHUNKS)
            def _():
                idx_start(c + 2, b)

            def add_body(s, _):
                for j in range(D // LANES):
                    sl = pl.ds(j * LANES, LANES)
                    e = enc_v[s, sl]
                    for q in range(SEQ_PER_CHUNK):
                        rows_v[b, q * SEQ + s, sl] += e
                return ()

            lax.fori_loop(0, SEQ, add_body, (), unroll=False)

            out_start(c, b)
        return ()

    lax.fori_loop(0, N_CHUNKS // 2, pair_body, (), unroll=False)

    out_wait((N_CHUNKS - 1) % 2)


@jax.jit
def kernel(x, table):
    x2d = x.reshape(X_ROWS, IDX_MINOR).astype(jnp.int32)
    enc = jnp.asarray(_ENC)
    mesh = plsc.VectorSubcoreMesh(core_axis_name="c", subcore_axis_name="s")
    out = pl.kernel(
        _body,
        out_type=jax.ShapeDtypeStruct((TOKENS, D), jnp.float32),
        mesh=mesh,
        compiler_params=pltpu.CompilerParams(use_tc_tiling_on_sc=False),
        scratch_types=[
            pltpu.VMEM((2, IDX_ROWS, IDX_MINOR), jnp.int32),
            pltpu.VMEM((2, CHUNK, D), jnp.float32),
            pltpu.VMEM((SEQ, D), jnp.float32),
            pltpu.SemaphoreType.DMA((2,)),
            pltpu.SemaphoreType.DMA((2,)),
            pltpu.SemaphoreType.DMA((2,)),
        ],
    )(x2d, table, enc)
    return out.reshape(BATCH, SEQ, D)


# speedup vs baseline: 2.5354x; 1.1063x over previous
import functools

import jax
import jax.numpy as jnp
import numpy as np
from jax import lax
from jax.experimental import pallas as pl
from jax.experimental.pallas import tpu as pltpu
from jax.experimental.pallas import tpu_sc as plsc

VOCAB = 1000000
D = 64
BATCH = 4096
SEQ = 200

NC = 2
NS = 16
NW = NC * NS
LANES = 16

BT = BATCH // NW
DT = D // 8
LB = BT // LANES
DJ = D // LANES


def _compute_encoding(max_len, d):
    enc = np.zeros((max_len, d), dtype=np.float32)
    pos = np.arange(0, max_len, dtype=np.float32)
    for i in range(d // 2):
        enc[:, 2 * i] = np.sin(pos / 10000 ** (2 * i / d))
        enc[:, 2 * i + 1] = np.cos(pos / 10000 ** (2 * i / d))
    return enc


_ENC = _compute_encoding(SEQ, D)


def _lane_shuffle(vec, idx):
    return lax.gather(
        vec, idx[:, None],
        lax.GatherDimensionNumbers(
            offset_dims=(), collapsed_slice_dims=(0,), start_index_map=(0,)),
        slice_sizes=(1,),
        mode=lax.GatherScatterMode.PROMISE_IN_BOUNDS)


def _body(xt_hbm, tab_hbm, enc_hbm, out_hbm,
          idx_v, emb_v, out_v, enc_v, idx_sem, gat_sem, out_sem):
    w = lax.axis_index("s") * NC + lax.axis_index("c")

    pltpu.sync_copy(enc_hbm, enc_v)

    def idx_start(s, b):
        pltpu.make_async_copy(
            xt_hbm.at[s, pl.ds(w * BT, BT)], idx_v.at[b], idx_sem.at[b]).start()

    def idx_wait(b):
        pltpu.make_async_copy(
            xt_hbm.at[0, pl.ds(0, BT)], idx_v.at[b], idx_sem.at[b]).wait()

    def gather_start(b):
        pltpu.make_async_copy(
            tab_hbm.at[idx_v.at[b]], emb_v.at[b], gat_sem.at[b]).start()

    def gather_wait(b):
        pltpu.make_async_copy(
            tab_hbm.at[pl.ds(0, BT)], emb_v.at[b], gat_sem.at[b]).wait()

    def out_start(s, b):
        pltpu.make_async_copy(
            out_v.at[b], out_hbm.at[s, :, w], out_sem.at[b]).start()

    def out_wait(b):
        pltpu.make_async_copy(
            out_v.at[b], out_hbm.at[0, :, 0], out_sem.at[b]).wait()

    iota = lax.iota(jnp.int32, LANES)
    rows = [lb * LANES + iota for lb in range(LB)]
    three = jnp.int32(3)
    seven = jnp.int32(7)
    fifteen = jnp.int32(LANES - 1)

    def compute(s, b):
        evecs = [enc_v[s, pl.ds(LANES * dj, LANES)] for dj in range(DJ)]

        def k_body(k, _):
            rot = lax.bitwise_and(iota + k, fifteen)
            for dj in range(DJ):
                dval = rot + (LANES * dj)
                dtv = lax.shift_right_logical(dval, three)
                d8v = lax.bitwise_and(dval, seven)
                e_rot = _lane_shuffle(evecs[dj], rot)
                for lb in range(LB):
                    g = plsc.load_gather(emb_v.at[b], [rows[lb], dval])
                    plsc.store_scatter(
                        out_v.at[b], [dtv, d8v, rows[lb]], g + e_rot)
            return ()

        lax.fori_loop(0, LANES, k_body, (), unroll=False)

    idx_start(0, 0)
    idx_wait(0)
    gather_start(0)
    idx_start(1, 1)

    def pair_body(s2, _):
        for b in (0, 1):
            s = 2 * s2 + b
            nb = 1 - b

            @pl.when(s >= 2)
            def _():
                out_wait(b)

            gather_wait(b)

            @pl.when(s + 1 < SEQ)
            def _():
                idx_wait(nb)
                gather_start(nb)

            @pl.when(s + 2 < SEQ)
            def _():
                idx_start(s + 2, b)

            compute(s, b)
            out_start(s, b)
        return ()

    lax.fori_loop(0, SEQ // 2, pair_body, (), unroll=False)

    out_wait(0)
    out_wait(1)


@jax.jit
def kernel(x, table):
    xt = x.T
    enc = jnp.asarray(_ENC)
    mesh = plsc.VectorSubcoreMesh(core_axis_name="c", subcore_axis_name="s")
    out5d = pl.kernel(
        _body,
        out_type=jax.ShapeDtypeStruct((SEQ, DT, NW, 8, BT), jnp.float32),
        mesh=mesh,
        compiler_params=pltpu.CompilerParams(
            use_tc_tiling_on_sc=False, needs_layout_passes=False),
        scratch_types=[
            pltpu.VMEM((2, BT), jnp.int32),
            pltpu.VMEM((2, BT, D), jnp.float32),
            pltpu.VMEM((2, DT, 8, BT), jnp.float32),
            pltpu.VMEM((SEQ, D), jnp.float32),
            pltpu.SemaphoreType.DMA((2,)),
            pltpu.SemaphoreType.DMA((2,)),
            pltpu.SemaphoreType.DMA((2,)),
        ],
    )(xt, table, enc)
    return out5d.transpose(2, 4, 0, 1, 3).reshape(BATCH, SEQ, D)


# speedup vs baseline: 2.6436x; 1.0427x over previous
import functools

import jax
import jax.numpy as jnp
import numpy as np
from jax import lax
from jax.experimental import pallas as pl
from jax.experimental.pallas import tpu as pltpu
from jax.experimental.pallas import tpu_sc as plsc

VOCAB = 1000000
D = 64
BATCH = 4096
SEQ = 200

NC = 2
NS = 16
NW = NC * NS
LANES = 16

BT = BATCH // NW
DT = D // 8
LB = BT // LANES
DJ = D // LANES


def _compute_encoding(max_len, d):
    enc = np.zeros((max_len, d), dtype=np.float32)
    pos = np.arange(0, max_len, dtype=np.float32)
    for i in range(d // 2):
        enc[:, 2 * i] = np.sin(pos / 10000 ** (2 * i / d))
        enc[:, 2 * i + 1] = np.cos(pos / 10000 ** (2 * i / d))
    return enc


_ENC = _compute_encoding(SEQ, D)


def _lane_shuffle(vec, idx):
    return lax.gather(
        vec, idx[:, None],
        lax.GatherDimensionNumbers(
            offset_dims=(), collapsed_slice_dims=(0,), start_index_map=(0,)),
        slice_sizes=(1,),
        mode=lax.GatherScatterMode.PROMISE_IN_BOUNDS)


def _body(xt_hbm, tab_hbm, enc_hbm, out_hbm,
          idx_v, emb_v, out_v, enc_v, idx_sem, gat_sem, out_sem):
    w = lax.axis_index("s") * NC + lax.axis_index("c")

    pltpu.sync_copy(enc_hbm, enc_v)

    def idx_start(s, b):
        pltpu.make_async_copy(
            xt_hbm.at[s, pl.ds(w * BT, BT)], idx_v.at[b], idx_sem.at[b]).start()

    def idx_wait(b):
        pltpu.make_async_copy(
            xt_hbm.at[0, pl.ds(0, BT)], idx_v.at[b], idx_sem.at[b]).wait()

    def gather_start(b):
        pltpu.make_async_copy(
            tab_hbm.at[idx_v.at[b]], emb_v.at[b], gat_sem.at[b]).start()

    def gather_wait(b):
        pltpu.make_async_copy(
            tab_hbm.at[pl.ds(0, BT)], emb_v.at[b], gat_sem.at[b]).wait()

    def out_start(s, b):
        pltpu.make_async_copy(
            out_v.at[b], out_hbm.at[s, :, w], out_sem.at[b]).start()

    def out_wait(b):
        pltpu.make_async_copy(
            out_v.at[b], out_hbm.at[0, :, 0], out_sem.at[b]).wait()

    iota = lax.iota(jnp.int32, LANES)
    rows = [lb * LANES + iota for lb in range(LB)]
    three = jnp.int32(3)
    seven = jnp.int32(7)
    fifteen = jnp.int32(LANES - 1)

    def compute(s, b):
        evecs = [enc_v[s, pl.ds(LANES * dj, LANES)] for dj in range(DJ)]

        def k_body(k, _):
            rot = lax.bitwise_and(iota + k, fifteen)
            for dj in range(DJ):
                dval = rot + (LANES * dj)
                dtv = lax.shift_right_logical(dval, three)
                d8v = lax.bitwise_and(dval, seven)
                e_rot = _lane_shuffle(evecs[dj], rot)
                for lb in range(LB):
                    g = plsc.load_gather(emb_v.at[b], [rows[lb], dval])
                    plsc.store_scatter(
                        out_v.at[b], [dtv, d8v, rows[lb]], g + e_rot)
            return ()

        lax.fori_loop(0, LANES, k_body, (), unroll=4)

    idx_start(0, 0)
    idx_wait(0)
    gather_start(0)
    idx_start(1, 1)

    def pair_body(s2, _):
        for b in (0, 1):
            s = 2 * s2 + b
            nb = 1 - b

            @pl.when(s >= 2)
            def _():
                out_wait(b)

            gather_wait(b)

            @pl.when(s + 1 < SEQ)
            def _():
                idx_wait(nb)
                gather_start(nb)

            @pl.when(s + 2 < SEQ)
            def _():
                idx_start(s + 2, b)

            compute(s, b)
            out_start(s, b)
        return ()

    lax.fori_loop(0, SEQ // 2, pair_body, (), unroll=False)

    out_wait(0)
    out_wait(1)


@jax.jit
def kernel(x, table):
    xt = x.T
    enc = jnp.asarray(_ENC)
    mesh = plsc.VectorSubcoreMesh(core_axis_name="c", subcore_axis_name="s")
    out5d = pl.kernel(
        _body,
        out_type=jax.ShapeDtypeStruct((SEQ, DT, NW, 8, BT), jnp.float32),
        mesh=mesh,
        compiler_params=pltpu.CompilerParams(
            use_tc_tiling_on_sc=False, needs_layout_passes=False),
        scratch_types=[
            pltpu.VMEM((2, BT), jnp.int32),
            pltpu.VMEM((2, BT, D), jnp.float32),
            pltpu.VMEM((2, DT, 8, BT), jnp.float32),
            pltpu.VMEM((SEQ, D), jnp.float32),
            pltpu.SemaphoreType.DMA((2,)),
            pltpu.SemaphoreType.DMA((2,)),
            pltpu.SemaphoreType.DMA((2,)),
        ],
    )(xt, table, enc)
    return out5d.transpose(2, 4, 0, 1, 3).reshape(BATCH, SEQ, D)
